# Initial kernel scaffold; baseline (speedup 1.0000x reference)
import jax
import jax.numpy as jnp
from jax.experimental import pallas as pl


def kernel(x, edge_index, W1, b1, W2, b2, g1, bt1, g2, bt2, mW1, mb1, mW2, mb2, mW3, mb3):
    raise NotImplementedError("write your pallas kernel here")


# speedup vs baseline: 32.2701x; 32.2701x over previous
import functools

import jax
import jax.numpy as jnp
from jax import lax
from jax.experimental import pallas as pl
from jax.experimental.pallas import tpu as pltpu
from jax.experimental.pallas import tpu_sc as plsc

N = 10000
D_IN = 128
H = 64

NC = 2
NS = 16
NW = NC * NS

N_PAD = 10240
ROWS_PER_SUB = N_PAD // NS
CHUNK = 128
CHUNKS_PER_W = 80
EW = CHUNK * CHUNKS_PER_W
E_PAD = EW * NW

_mesh = plsc.VectorSubcoreMesh(core_axis_name="c", subcore_axis_name="s")
_sc_params = pltpu.CompilerParams(use_tc_tiling_on_sc=False)


@functools.partial(
    pl.kernel,
    out_type=jax.ShapeDtypeStruct((NC, N_PAD, 16), jnp.float32),
    mesh=_mesh,
    scratch_types=[
        pltpu.VMEM((CHUNKS_PER_W, CHUNK), jnp.int32),
        pltpu.VMEM((CHUNK, 16), jnp.float32),
        pltpu.VMEM_SHARED((N_PAD, 16), jnp.float32),
        pltpu.SemaphoreType.DMA,
    ],
    compiler_params=_sc_params,
)
def _deg_kernel(dst_hbm, zeros_hbm, ones_hbm, out_hbm, didx, ones_v, acc, sem):
    c = lax.axis_index("c")
    s = lax.axis_index("s")
    wid = s * NC + c
    rbase = s * ROWS_PER_SUB
    pltpu.sync_copy(ones_hbm, ones_v)
    pltpu.sync_copy(dst_hbm.at[pl.ds(wid * CHUNKS_PER_W, CHUNKS_PER_W)], didx)
    pltpu.sync_copy(zeros_hbm.at[pl.ds(rbase, ROWS_PER_SUB)],
                    acc.at[pl.ds(rbase, ROWS_PER_SUB)])
    plsc.subcore_barrier()

    def body(j, carry):
        pltpu.sync_copy(ones_v, acc.at[didx.at[j]], add=True)
        return carry

    lax.fori_loop(0, CHUNKS_PER_W, body, 0)
    plsc.subcore_barrier()
    pltpu.sync_copy(acc.at[pl.ds(rbase, ROWS_PER_SUB)],
                    out_hbm.at[c, pl.ds(rbase, ROWS_PER_SUB)])


@functools.partial(
    pl.kernel,
    out_type=jax.ShapeDtypeStruct((NC, N_PAD, H), jnp.float32),
    mesh=_mesh,
    scratch_types=[
        pltpu.VMEM((CHUNKS_PER_W, CHUNK), jnp.int32),
        pltpu.VMEM((CHUNKS_PER_W, CHUNK), jnp.int32),
        pltpu.VMEM((CHUNK, H), jnp.float32),
        pltpu.VMEM((CHUNK, H), jnp.float32),
        pltpu.VMEM_SHARED((N_PAD, H), jnp.float32),
        pltpu.SemaphoreType.DMA,
        pltpu.SemaphoreType.DMA,
    ],
    compiler_params=_sc_params,
)
def _conv_kernel(hs_hbm, src_hbm, dst_hbm, zeros_hbm, out_hbm,
                 sidx, didx, rows_a, rows_b, acc, sem_a, sem_b):
    c = lax.axis_index("c")
    s = lax.axis_index("s")
    wid = s * NC + c
    rbase = s * ROWS_PER_SUB
    pltpu.sync_copy(src_hbm.at[pl.ds(wid * CHUNKS_PER_W, CHUNKS_PER_W)], sidx)
    pltpu.sync_copy(dst_hbm.at[pl.ds(wid * CHUNKS_PER_W, CHUNKS_PER_W)], didx)
    pltpu.sync_copy(zeros_hbm.at[pl.ds(rbase, ROWS_PER_SUB)],
                    acc.at[pl.ds(rbase, ROWS_PER_SUB)])
    plsc.subcore_barrier()

    def body(i, carry):
        j0 = i * 2
        j1 = j0 + 1
        cp_a = pltpu.async_copy(hs_hbm.at[sidx.at[j0]], rows_a, sem_a)
        cp_b = pltpu.async_copy(hs_hbm.at[sidx.at[j1]], rows_b, sem_b)
        cp_a.wait()
        pltpu.sync_copy(rows_a, acc.at[didx.at[j0]], add=True)
        cp_b.wait()
        pltpu.sync_copy(rows_b, acc.at[didx.at[j1]], add=True)
        return carry

    lax.fori_loop(0, CHUNKS_PER_W // 2, body, 0)
    plsc.subcore_barrier()
    pltpu.sync_copy(acc.at[pl.ds(rbase, ROWS_PER_SUB)],
                    out_hbm.at[c, pl.ds(rbase, ROWS_PER_SUB)])


_BN_C = 1.0 / (1.0 + 1e-5) ** 0.5


def _dinv_from(degp_ref):
    deg = (degp_ref[0, :N, 0:1] + degp_ref[1, :N, 0:1]) + 1.0
    return lax.rsqrt(deg)


def _tc1_body(x_ref, w1_ref, degp_ref, hs_ref):
    y = jnp.dot(x_ref[...], w1_ref[...], preferred_element_type=jnp.float32)
    hs_ref[...] = y * _dinv_from(degp_ref)


def _tc2_body(p_ref, hs_ref, degp_ref, w2_ref, b1_ref, g1_ref, bt1_ref,
              hs2_ref):
    dinv = _dinv_from(degp_ref)
    agg = dinv * (p_ref[0, :N, :] + p_ref[1, :N, :] + hs_ref[...]) + b1_ref[...]
    h1 = jax.nn.relu(agg * (g1_ref[...] * _BN_C) + bt1_ref[...])
    y2 = jnp.dot(h1, w2_ref[...], preferred_element_type=jnp.float32)
    hs2_ref[...] = y2 * dinv


def _tc3_body(p_ref, hs_ref, degp_ref, b2_ref, g2_ref, bt2_ref,
              mw1_ref, mb1_ref, mw2_ref, mb2_ref, mw3_ref, mb3_ref, out_ref):
    dinv = _dinv_from(degp_ref)
    agg = dinv * (p_ref[0, :N, :] + p_ref[1, :N, :] + hs_ref[...]) + b2_ref[...]
    h2 = jax.nn.relu(agg * (g2_ref[...] * _BN_C) + bt2_ref[...])
    m1 = jax.nn.relu(
        jnp.dot(h2, mw1_ref[...], preferred_element_type=jnp.float32)
        + mb1_ref[...])
    m2 = jax.nn.relu(
        jnp.dot(m1, mw2_ref[...], preferred_element_type=jnp.float32)
        + mb2_ref[...])
    z = (jnp.dot(m2, mw3_ref[...], preferred_element_type=jnp.float32)
         + mb3_ref[...])
    out_ref[...] = jax.nn.sigmoid(z)


_tc1 = pl.pallas_call(
    _tc1_body, out_shape=jax.ShapeDtypeStruct((N, H), jnp.float32))
_tc2 = pl.pallas_call(
    _tc2_body, out_shape=jax.ShapeDtypeStruct((N, H), jnp.float32))
_tc3 = pl.pallas_call(
    _tc3_body, out_shape=jax.ShapeDtypeStruct((N, 1), jnp.float32))


def kernel(x, edge_index, W1, b1, W2, b2, g1, bt1, g2, bt2,
           mW1, mb1, mW2, mb2, mW3, mb3):
    E = edge_index.shape[1]
    pad = E_PAD - E
    pad_ar = jnp.arange(pad, dtype=jnp.int32)
    src_p = jnp.concatenate([edge_index[0], pad_ar % 1024])
    dst_p = jnp.concatenate([edge_index[1], N + pad_ar % (N_PAD - N)])
    src2d = src_p.reshape(E_PAD // CHUNK, CHUNK)
    dst2d = dst_p.reshape(E_PAD // CHUNK, CHUNK)

    zeros_d = jnp.zeros((N_PAD, 16), jnp.float32)
    zeros_c = jnp.zeros((N_PAD, H), jnp.float32)
    ones_r = jnp.ones((CHUNK, 16), jnp.float32)

    degp = _deg_kernel(dst2d, zeros_d, ones_r)
    hs1 = _tc1(x, W1, degp)
    p1 = _conv_kernel(hs1, src2d, dst2d, zeros_c)
    hs2 = _tc2(p1, hs1, degp, W2, b1, g1, bt1)
    p2 = _conv_kernel(hs2, src2d, dst2d, zeros_c)
    out = _tc3(p2, hs2, degp, b2, g2, bt2,
               mW1, mb1, mW2, mb2, mW3, mb3)
    return out.reshape(N)


# speedup vs baseline: 37.6993x; 1.1682x over previous
import functools

import jax
import jax.numpy as jnp
from jax import lax
from jax.experimental import pallas as pl
from jax.experimental.pallas import tpu as pltpu
from jax.experimental.pallas import tpu_sc as plsc

N = 10000
D_IN = 128
H = 64

NC = 2
NS = 16
NW = NC * NS

N_PAD = 10240
ROWS_PER_SUB = N_PAD // NS
CHUNK = 128
CHUNKS_PER_W = 80
EW = CHUNK * CHUNKS_PER_W
E_PAD = EW * NW

_mesh = plsc.VectorSubcoreMesh(core_axis_name="c", subcore_axis_name="s")
_sc_params = pltpu.CompilerParams(use_tc_tiling_on_sc=False)


@functools.partial(
    pl.kernel,
    out_type=jax.ShapeDtypeStruct((NC, N_PAD, 16), jnp.float32),
    mesh=_mesh,
    scratch_types=[
        pltpu.VMEM((CHUNKS_PER_W, CHUNK), jnp.int32),
        pltpu.VMEM((CHUNK, 16), jnp.float32),
        pltpu.VMEM_SHARED((N_PAD, 16), jnp.float32),
        [pltpu.SemaphoreType.DMA] * 8,
    ],
    compiler_params=_sc_params,
)
def _deg_kernel(dst_hbm, zeros_hbm, ones_hbm, out_hbm, didx, ones_v, acc, sems):
    c = lax.axis_index("c")
    s = lax.axis_index("s")
    wid = s * NC + c
    rbase = s * ROWS_PER_SUB
    pltpu.sync_copy(ones_hbm, ones_v)
    pltpu.sync_copy(dst_hbm.at[pl.ds(wid * CHUNKS_PER_W, CHUNKS_PER_W)], didx)
    pltpu.sync_copy(zeros_hbm.at[pl.ds(rbase, ROWS_PER_SUB)],
                    acc.at[pl.ds(rbase, ROWS_PER_SUB)])
    plsc.subcore_barrier()

    def body(k, carry):
        base = k * 8
        cps = [pltpu.async_copy(ones_v, acc.at[didx.at[base + b]], sems[b],
                                add=True) for b in range(8)]
        for cp in cps:
            cp.wait()
        return carry

    lax.fori_loop(0, CHUNKS_PER_W // 8, body, 0)
    plsc.subcore_barrier()
    pltpu.sync_copy(acc.at[pl.ds(rbase, ROWS_PER_SUB)],
                    out_hbm.at[c, pl.ds(rbase, ROWS_PER_SUB)])


@functools.partial(
    pl.kernel,
    out_type=jax.ShapeDtypeStruct((NC, N_PAD, H), jnp.float32),
    mesh=_mesh,
    scratch_types=[
        pltpu.VMEM((CHUNKS_PER_W, CHUNK), jnp.int32),
        pltpu.VMEM((CHUNKS_PER_W, CHUNK), jnp.int32),
        pltpu.VMEM((8, CHUNK, H), jnp.float32),
        pltpu.VMEM_SHARED((N_PAD, H), jnp.float32),
        [pltpu.SemaphoreType.DMA] * 8,
        [pltpu.SemaphoreType.DMA] * 8,
    ],
    compiler_params=_sc_params,
)
def _conv_kernel(hs_hbm, src_hbm, dst_hbm, zeros_hbm, out_hbm,
                 sidx, didx, rows, acc, gsems, ssems):
    c = lax.axis_index("c")
    s = lax.axis_index("s")
    wid = s * NC + c
    rbase = s * ROWS_PER_SUB
    pltpu.sync_copy(src_hbm.at[pl.ds(wid * CHUNKS_PER_W, CHUNKS_PER_W)], sidx)
    pltpu.sync_copy(dst_hbm.at[pl.ds(wid * CHUNKS_PER_W, CHUNKS_PER_W)], didx)
    pltpu.sync_copy(zeros_hbm.at[pl.ds(rbase, ROWS_PER_SUB)],
                    acc.at[pl.ds(rbase, ROWS_PER_SUB)])
    plsc.subcore_barrier()

    def gather(j, b):
        return pltpu.async_copy(hs_hbm.at[sidx.at[j]], rows.at[b], gsems[b])

    def scatter(j, b):
        return pltpu.async_copy(rows.at[b], acc.at[didx.at[j]], ssems[b],
                                add=True)

    def body(k, carry):
        base = k * 8
        cg = [gather(base + b, b) for b in range(4)]
        cs = []
        for b in range(4):
            cg[b].wait()
            cs.append(scatter(base + b, b))
        cg2 = [gather(base + 4 + b, 4 + b) for b in range(4)]
        for cp in cs:
            cp.wait()
        cs2 = []
        for b in range(4):
            cg2[b].wait()
            cs2.append(scatter(base + 4 + b, 4 + b))
        for cp in cs2:
            cp.wait()
        return carry

    lax.fori_loop(0, CHUNKS_PER_W // 8, body, 0)
    plsc.subcore_barrier()
    pltpu.sync_copy(acc.at[pl.ds(rbase, ROWS_PER_SUB)],
                    out_hbm.at[c, pl.ds(rbase, ROWS_PER_SUB)])


_BN_C = 1.0 / (1.0 + 1e-5) ** 0.5


def _dinv_from(degp_ref):
    deg = (degp_ref[0, :N, 0:1] + degp_ref[1, :N, 0:1]) + 1.0
    return lax.rsqrt(deg)


def _tc1_body(x_ref, w1_ref, degp_ref, hs_ref):
    y = jnp.dot(x_ref[...], w1_ref[...], preferred_element_type=jnp.float32)
    hs_ref[...] = y * _dinv_from(degp_ref)


def _tc2_body(p_ref, hs_ref, degp_ref, w2_ref, b1_ref, g1_ref, bt1_ref,
              hs2_ref):
    dinv = _dinv_from(degp_ref)
    agg = dinv * (p_ref[0, :N, :] + p_ref[1, :N, :] + hs_ref[...]) + b1_ref[...]
    h1 = jax.nn.relu(agg * (g1_ref[...] * _BN_C) + bt1_ref[...])
    y2 = jnp.dot(h1, w2_ref[...], preferred_element_type=jnp.float32)
    hs2_ref[...] = y2 * dinv


def _tc3_body(p_ref, hs_ref, degp_ref, b2_ref, g2_ref, bt2_ref,
              mw1_ref, mb1_ref, mw2_ref, mb2_ref, mw3_ref, mb3_ref, out_ref):
    dinv = _dinv_from(degp_ref)
    agg = dinv * (p_ref[0, :N, :] + p_ref[1, :N, :] + hs_ref[...]) + b2_ref[...]
    h2 = jax.nn.relu(agg * (g2_ref[...] * _BN_C) + bt2_ref[...])
    m1 = jax.nn.relu(
        jnp.dot(h2, mw1_ref[...], preferred_element_type=jnp.float32)
        + mb1_ref[...])
    m2 = jax.nn.relu(
        jnp.dot(m1, mw2_ref[...], preferred_element_type=jnp.float32)
        + mb2_ref[...])
    z = (jnp.dot(m2, mw3_ref[...], preferred_element_type=jnp.float32)
         + mb3_ref[...])
    out_ref[...] = jax.nn.sigmoid(z)


_tc1 = pl.pallas_call(
    _tc1_body, out_shape=jax.ShapeDtypeStruct((N, H), jnp.float32))
_tc2 = pl.pallas_call(
    _tc2_body, out_shape=jax.ShapeDtypeStruct((N, H), jnp.float32))
_tc3 = pl.pallas_call(
    _tc3_body, out_shape=jax.ShapeDtypeStruct((N, 1), jnp.float32))


def kernel(x, edge_index, W1, b1, W2, b2, g1, bt1, g2, bt2,
           mW1, mb1, mW2, mb2, mW3, mb3):
    E = edge_index.shape[1]
    pad = E_PAD - E
    pad_ar = jnp.arange(pad, dtype=jnp.int32)
    src_p = jnp.concatenate([edge_index[0], pad_ar % 1024])
    dst_p = jnp.concatenate([edge_index[1], N + pad_ar % (N_PAD - N)])
    src2d = src_p.reshape(E_PAD // CHUNK, CHUNK)
    dst2d = dst_p.reshape(E_PAD // CHUNK, CHUNK)

    zeros_d = jnp.zeros((N_PAD, 16), jnp.float32)
    zeros_c = jnp.zeros((N_PAD, H), jnp.float32)
    ones_r = jnp.ones((CHUNK, 16), jnp.float32)

    degp = _deg_kernel(dst2d, zeros_d, ones_r)
    hs1 = _tc1(x, W1, degp)
    p1 = _conv_kernel(hs1, src2d, dst2d, zeros_c)
    hs2 = _tc2(p1, hs1, degp, W2, b1, g1, bt1)
    p2 = _conv_kernel(hs2, src2d, dst2d, zeros_c)
    out = _tc3(p2, hs2, degp, b2, g2, bt2,
               mW1, mb1, mW2, mb2, mW3, mb3)
    return out.reshape(N)


# speedup vs baseline: 41.3969x; 1.0981x over previous
import functools

import jax
import jax.numpy as jnp
from jax import lax
from jax.experimental import pallas as pl
from jax.experimental.pallas import tpu as pltpu
from jax.experimental.pallas import tpu_sc as plsc

N = 10000
D_IN = 128
H = 64

NC = 2
NS = 16
NW = NC * NS

N_PAD = 10240
ROWS_PER_SUB = N_PAD // NS
CHUNK = 128
CHUNKS_PER_W = 80
EW = CHUNK * CHUNKS_PER_W
E_PAD = EW * NW

_mesh = plsc.VectorSubcoreMesh(core_axis_name="c", subcore_axis_name="s")
_sc_params = pltpu.CompilerParams(use_tc_tiling_on_sc=False)


@functools.partial(
    pl.kernel,
    out_type=jax.ShapeDtypeStruct((NC, N_PAD, 16), jnp.float32),
    mesh=_mesh,
    scratch_types=[
        pltpu.VMEM((CHUNKS_PER_W, CHUNK), jnp.int32),
        pltpu.VMEM((CHUNK, 16), jnp.float32),
        pltpu.VMEM_SHARED((N_PAD, 16), jnp.float32),
        [pltpu.SemaphoreType.DMA] * 8,
    ],
    compiler_params=_sc_params,
)
def _deg_kernel(eidx_hbm, zeros_hbm, ones_hbm, out_hbm, didx, ones_v, acc, sems):
    c = lax.axis_index("c")
    s = lax.axis_index("s")
    wid = s * NC + c
    rbase = s * ROWS_PER_SUB
    pltpu.sync_copy(ones_hbm, ones_v)
    pltpu.sync_copy(eidx_hbm.at[1, pl.ds(wid * CHUNKS_PER_W, CHUNKS_PER_W)],
                    didx)
    pltpu.sync_copy(zeros_hbm.at[pl.ds(rbase, ROWS_PER_SUB)],
                    acc.at[pl.ds(rbase, ROWS_PER_SUB)])
    plsc.subcore_barrier()

    def body(k, carry):
        base = k * 8
        cps = [pltpu.async_copy(ones_v, acc.at[didx.at[base + b]], sems[b],
                                add=True) for b in range(8)]
        for cp in cps:
            cp.wait()
        return carry

    lax.fori_loop(0, CHUNKS_PER_W // 8, body, 0)
    plsc.subcore_barrier()
    pltpu.sync_copy(acc.at[pl.ds(rbase, ROWS_PER_SUB)],
                    out_hbm.at[c, pl.ds(rbase, ROWS_PER_SUB)])


@functools.partial(
    pl.kernel,
    out_type=jax.ShapeDtypeStruct((NC, N_PAD, H), jnp.float32),
    mesh=_mesh,
    scratch_types=[
        pltpu.VMEM((CHUNKS_PER_W, CHUNK), jnp.int32),
        pltpu.VMEM((CHUNKS_PER_W, CHUNK), jnp.int32),
        pltpu.VMEM((8, CHUNK, H), jnp.float32),
        pltpu.VMEM_SHARED((N_PAD, H), jnp.float32),
        [pltpu.SemaphoreType.DMA] * 8,
        [pltpu.SemaphoreType.DMA] * 8,
    ],
    compiler_params=_sc_params,
)
def _conv_kernel(hs_hbm, eidx_hbm, zeros_hbm, out_hbm,
                 sidx, didx, rows, acc, gsems, ssems):
    c = lax.axis_index("c")
    s = lax.axis_index("s")
    wid = s * NC + c
    rbase = s * ROWS_PER_SUB
    pltpu.sync_copy(eidx_hbm.at[0, pl.ds(wid * CHUNKS_PER_W, CHUNKS_PER_W)],
                    sidx)
    pltpu.sync_copy(eidx_hbm.at[1, pl.ds(wid * CHUNKS_PER_W, CHUNKS_PER_W)],
                    didx)
    pltpu.sync_copy(zeros_hbm.at[pl.ds(rbase, ROWS_PER_SUB)],
                    acc.at[pl.ds(rbase, ROWS_PER_SUB)])
    plsc.subcore_barrier()

    def gather(j, b):
        return pltpu.async_copy(hs_hbm.at[sidx.at[j]], rows.at[b], gsems[b])

    def scatter(j, b):
        return pltpu.async_copy(rows.at[b], acc.at[didx.at[j]], ssems[b],
                                add=True)

    cg_pro = [gather(b, b) for b in range(4)]

    def body(k, carry):
        base = k * 8
        cs = []
        for b in range(4):
            pltpu.make_async_copy(hs_hbm.at[sidx.at[base + b]], rows.at[b],
                                  gsems[b]).wait()
            cs.append(scatter(base + b, b))
        cg2 = [gather(base + 4 + b, 4 + b) for b in range(4)]
        for cp in cs:
            cp.wait()
        cs2 = []
        for b in range(4):
            cg2[b].wait()
            cs2.append(scatter(base + 4 + b, 4 + b))
        for b in range(4):
            jn = jnp.minimum(base + 8 + b, CHUNKS_PER_W - 1)
            gather(jn, b)
        for cp in cs2:
            cp.wait()
        return carry

    lax.fori_loop(0, CHUNKS_PER_W // 8, body, 0)
    for b in range(4):
        pltpu.make_async_copy(hs_hbm.at[sidx.at[CHUNKS_PER_W - 1]],
                              rows.at[b], gsems[b]).wait()
    plsc.subcore_barrier()
    pltpu.sync_copy(acc.at[pl.ds(rbase, ROWS_PER_SUB)],
                    out_hbm.at[c, pl.ds(rbase, ROWS_PER_SUB)])


_BN_C = 1.0 / (1.0 + 1e-5) ** 0.5


def _dinv_from(degp_ref):
    deg = (degp_ref[0, :N, 0:1] + degp_ref[1, :N, 0:1]) + 1.0
    return lax.rsqrt(deg)


def _tc1_body(x_ref, w1_ref, degp_ref, hs_ref):
    y = jnp.dot(x_ref[...], w1_ref[...], preferred_element_type=jnp.float32)
    hs_ref[...] = y * _dinv_from(degp_ref)


def _tc2_body(p_ref, hs_ref, degp_ref, w2_ref, b1_ref, g1_ref, bt1_ref,
              hs2_ref):
    dinv = _dinv_from(degp_ref)
    agg = dinv * (p_ref[0, :N, :] + p_ref[1, :N, :] + hs_ref[...]) + b1_ref[...]
    h1 = jax.nn.relu(agg * (g1_ref[...] * _BN_C) + bt1_ref[...])
    y2 = jnp.dot(h1, w2_ref[...], preferred_element_type=jnp.float32)
    hs2_ref[...] = y2 * dinv


def _tc3_body(p_ref, hs_ref, degp_ref, b2_ref, g2_ref, bt2_ref,
              mw1_ref, mb1_ref, mw2_ref, mb2_ref, mw3_ref, mb3_ref, out_ref):
    dinv = _dinv_from(degp_ref)
    agg = dinv * (p_ref[0, :N, :] + p_ref[1, :N, :] + hs_ref[...]) + b2_ref[...]
    h2 = jax.nn.relu(agg * (g2_ref[...] * _BN_C) + bt2_ref[...])
    m1 = jax.nn.relu(
        jnp.dot(h2, mw1_ref[...], preferred_element_type=jnp.float32)
        + mb1_ref[...])
    m2 = jax.nn.relu(
        jnp.dot(m1, mw2_ref[...], preferred_element_type=jnp.float32)
        + mb2_ref[...])
    z = (jnp.dot(m2, mw3_ref[...], preferred_element_type=jnp.float32)
         + mb3_ref[...])
    out_ref[...] = jax.nn.sigmoid(z)


_tc1 = pl.pallas_call(
    _tc1_body, out_shape=jax.ShapeDtypeStruct((N, H), jnp.float32))
_tc2 = pl.pallas_call(
    _tc2_body, out_shape=jax.ShapeDtypeStruct((N, H), jnp.float32))
_tc3 = pl.pallas_call(
    _tc3_body, out_shape=jax.ShapeDtypeStruct((N, 1), jnp.float32))


def kernel(x, edge_index, W1, b1, W2, b2, g1, bt1, g2, bt2,
           mW1, mb1, mW2, mb2, mW3, mb3):
    E = edge_index.shape[1]
    pad = E_PAD - E
    pad_ar = jnp.arange(pad, dtype=jnp.int32)
    pads = jnp.stack([pad_ar % 1024, N + pad_ar % (N_PAD - N)])
    eidx3 = jnp.concatenate([edge_index, pads], axis=1).reshape(
        2, E_PAD // CHUNK, CHUNK)

    zeros_d = jnp.zeros((N_PAD, 16), jnp.float32)
    zeros_c = jnp.zeros((N_PAD, H), jnp.float32)
    ones_r = jnp.ones((CHUNK, 16), jnp.float32)

    degp = _deg_kernel(eidx3, zeros_d, ones_r)
    hs1 = _tc1(x, W1, degp)
    p1 = _conv_kernel(hs1, eidx3, zeros_c)
    hs2 = _tc2(p1, hs1, degp, W2, b1, g1, bt1)
    p2 = _conv_kernel(hs2, eidx3, zeros_c)
    out = _tc3(p2, hs2, degp, b2, g2, bt2,
               mW1, mb1, mW2, mb2, mW3, mb3)
    return out.reshape(N)


# speedup vs baseline: 48.6957x; 1.1763x over previous
import functools

import jax
import jax.numpy as jnp
from jax import lax
from jax.experimental import pallas as pl
from jax.experimental.pallas import tpu as pltpu
from jax.experimental.pallas import tpu_sc as plsc

N = 10000
D_IN = 128
H = 64

NC = 2
NS = 16
NW = NC * NS

N_PAD = 10240
ROWS_PER_SUB = N_PAD // NS
CHUNK = 128
CHUNKS_PER_W = 80
EW = CHUNK * CHUNKS_PER_W
E_PAD = EW * NW

_mesh = plsc.VectorSubcoreMesh(core_axis_name="c", subcore_axis_name="s")
_sc_params = pltpu.CompilerParams(use_tc_tiling_on_sc=False)


@functools.partial(
    pl.kernel,
    out_type=jax.ShapeDtypeStruct((NC, N_PAD, 16), jnp.float32),
    mesh=_mesh,
    scratch_types=[
        pltpu.VMEM((CHUNKS_PER_W, CHUNK), jnp.int32),
        pltpu.VMEM((CHUNK, 16), jnp.float32),
        pltpu.VMEM_SHARED((N_PAD, 16), jnp.float32),
        [pltpu.SemaphoreType.DMA] * 8,
    ],
    compiler_params=_sc_params,
)
def _deg_kernel(eidx_hbm, zeros_hbm, ones_hbm, out_hbm, didx, ones_v, acc, sems):
    c = lax.axis_index("c")
    s = lax.axis_index("s")
    wid = s * NC + c
    rbase = s * ROWS_PER_SUB
    pltpu.sync_copy(ones_hbm, ones_v)
    pltpu.sync_copy(eidx_hbm.at[1, pl.ds(wid * CHUNKS_PER_W, CHUNKS_PER_W)],
                    didx)
    pltpu.sync_copy(zeros_hbm.at[pl.ds(rbase, ROWS_PER_SUB)],
                    acc.at[pl.ds(rbase, ROWS_PER_SUB)])
    plsc.subcore_barrier()

    def body(k, carry):
        base = k * 8
        cps = [pltpu.async_copy(ones_v, acc.at[didx.at[base + b]], sems[b],
                                add=True) for b in range(8)]
        for cp in cps:
            cp.wait()
        return carry

    lax.fori_loop(0, CHUNKS_PER_W // 8, body, 0)
    plsc.subcore_barrier()
    pltpu.sync_copy(acc.at[pl.ds(rbase, ROWS_PER_SUB)],
                    out_hbm.at[c, pl.ds(rbase, ROWS_PER_SUB)])


@functools.partial(
    pl.kernel,
    out_type=jax.ShapeDtypeStruct((NC, N_PAD, H), jnp.float32),
    mesh=_mesh,
    scratch_types=[
        pltpu.VMEM((CHUNKS_PER_W, CHUNK), jnp.int32),
        pltpu.VMEM((CHUNKS_PER_W, CHUNK), jnp.int32),
        pltpu.VMEM((8, CHUNK, H), jnp.float32),
        pltpu.VMEM_SHARED((N_PAD, H), jnp.float32),
        [pltpu.SemaphoreType.DMA] * 8,
        [pltpu.SemaphoreType.DMA] * 8,
    ],
    compiler_params=_sc_params,
)
def _conv_kernel(hs_hbm, eidx_hbm, zeros_hbm, out_hbm,
                 sidx, didx, rows, acc, gsems, ssems):
    c = lax.axis_index("c")
    s = lax.axis_index("s")
    wid = s * NC + c
    rbase = s * ROWS_PER_SUB
    pltpu.sync_copy(eidx_hbm.at[0, pl.ds(wid * CHUNKS_PER_W, CHUNKS_PER_W)],
                    sidx)
    pltpu.sync_copy(eidx_hbm.at[1, pl.ds(wid * CHUNKS_PER_W, CHUNKS_PER_W)],
                    didx)
    pltpu.sync_copy(zeros_hbm.at[pl.ds(rbase, ROWS_PER_SUB)],
                    acc.at[pl.ds(rbase, ROWS_PER_SUB)])
    plsc.subcore_barrier()

    def gather(j, b):
        return pltpu.async_copy(hs_hbm.at[sidx.at[j]], rows.at[b], gsems[b])

    def scatter(j, b):
        return pltpu.async_copy(rows.at[b], acc.at[didx.at[j]], ssems[b],
                                add=True)

    cg_pro = [gather(b, b) for b in range(4)]

    def body(k, carry):
        base = k * 8
        cs = []
        for b in range(4):
            pltpu.make_async_copy(hs_hbm.at[sidx.at[base + b]], rows.at[b],
                                  gsems[b]).wait()
            cs.append(scatter(base + b, b))
        cg2 = [gather(base + 4 + b, 4 + b) for b in range(4)]
        for cp in cs:
            cp.wait()
        cs2 = []
        for b in range(4):
            cg2[b].wait()
            cs2.append(scatter(base + 4 + b, 4 + b))
        for b in range(4):
            jn = jnp.minimum(base + 8 + b, CHUNKS_PER_W - 1)
            gather(jn, b)
        for cp in cs2:
            cp.wait()
        return carry

    lax.fori_loop(0, CHUNKS_PER_W // 8, body, 0)
    for b in range(4):
        pltpu.make_async_copy(hs_hbm.at[sidx.at[CHUNKS_PER_W - 1]],
                              rows.at[b], gsems[b]).wait()
    plsc.subcore_barrier()
    pltpu.sync_copy(acc.at[pl.ds(rbase, ROWS_PER_SUB)],
                    out_hbm.at[c, pl.ds(rbase, ROWS_PER_SUB)])


_BN_C = 1.0 / (1.0 + 1e-5) ** 0.5
HALF = N_PAD // 2
HI_N = N - HALF


def _dinv_from(degp_ref):
    deg_lo = degp_ref[0, :, 0:1] + degp_ref[1, :, 0:1] + 1.0
    deg_hi = degp_ref[0, :HI_N, 16:17] + degp_ref[1, :HI_N, 16:17] + 1.0
    return lax.rsqrt(deg_lo), lax.rsqrt(deg_hi)


def _tc1_body(x_ref, w1_ref, degp_ref, hs_ref):
    dlo, dhi = _dinv_from(degp_ref)
    y_lo = jnp.dot(x_ref[:HALF], w1_ref[...],
                   preferred_element_type=jnp.float32)
    y_hi = jnp.dot(x_ref[HALF:], w1_ref[...],
                   preferred_element_type=jnp.float32)
    hs_ref[:, :H] = y_lo * dlo
    hs_ref[:HI_N, H:] = y_hi * dhi
    hs_ref[HI_N:, H:] = jnp.zeros((HALF - HI_N, H), jnp.float32)


def _tc2_body(p_ref, hs_ref, degp_ref, w2_ref, b1_ref, g1_ref, bt1_ref,
              hs2_ref):
    dlo, dhi = _dinv_from(degp_ref)
    spk = p_ref[0] + p_ref[1] + hs_ref[...]
    c1 = g1_ref[...] * _BN_C
    h1_lo = jax.nn.relu((dlo * spk[:, :H] + b1_ref[...]) * c1 + bt1_ref[...])
    h1_hi = jax.nn.relu((dhi * spk[:HI_N, H:] + b1_ref[...]) * c1
                        + bt1_ref[...])
    y2_lo = jnp.dot(h1_lo, w2_ref[...], preferred_element_type=jnp.float32)
    y2_hi = jnp.dot(h1_hi, w2_ref[...], preferred_element_type=jnp.float32)
    hs2_ref[:, :H] = y2_lo * dlo
    hs2_ref[:HI_N, H:] = y2_hi * dhi
    hs2_ref[HI_N:, H:] = jnp.zeros((HALF - HI_N, H), jnp.float32)


def _tc3_body(p_ref, hs_ref, degp_ref, b2_ref, g2_ref, bt2_ref,
              mw1_ref, mb1_ref, mw2_ref, mb2_ref, mw3_ref, mb3_ref, out_ref):
    dlo, dhi = _dinv_from(degp_ref)
    spk = p_ref[0] + p_ref[1] + hs_ref[...]
    c2 = g2_ref[...] * _BN_C

    def head(h2):
        m1 = jax.nn.relu(
            jnp.dot(h2, mw1_ref[...], preferred_element_type=jnp.float32)
            + mb1_ref[...])
        m2 = jax.nn.relu(
            jnp.dot(m1, mw2_ref[...], preferred_element_type=jnp.float32)
            + mb2_ref[...])
        z = (jnp.dot(m2, mw3_ref[...], preferred_element_type=jnp.float32)
             + mb3_ref[...])
        return jax.nn.sigmoid(z)

    h2_lo = jax.nn.relu((dlo * spk[:, :H] + b2_ref[...]) * c2 + bt2_ref[...])
    h2_hi = jax.nn.relu((dhi * spk[:HI_N, H:] + b2_ref[...]) * c2
                        + bt2_ref[...])
    out_ref[:HALF, :] = head(h2_lo)
    out_ref[HALF:, :] = head(h2_hi)


_tc1 = pl.pallas_call(
    _tc1_body, out_shape=jax.ShapeDtypeStruct((HALF, 128), jnp.float32))
_tc2 = pl.pallas_call(
    _tc2_body, out_shape=jax.ShapeDtypeStruct((HALF, 128), jnp.float32))
_tc3 = pl.pallas_call(
    _tc3_body, out_shape=jax.ShapeDtypeStruct((N, 1), jnp.float32))


def kernel(x, edge_index, W1, b1, W2, b2, g1, bt1, g2, bt2,
           mW1, mb1, mW2, mb2, mW3, mb3):
    E = edge_index.shape[1]
    pad = E_PAD - E
    eperm = jnp.where(edge_index < HALF, 2 * edge_index,
                      2 * (edge_index - HALF) + 1)
    pad_ar = jnp.arange(pad, dtype=jnp.int32)
    pads = jnp.stack([pad_ar % 1024,
                      2 * (N - HALF) + 1 + 2 * (pad_ar % (N_PAD - N))])
    eidx3 = jnp.concatenate([eperm, pads], axis=1).reshape(
        2, E_PAD // CHUNK, CHUNK)

    zeros_d = jnp.zeros((N_PAD, 16), jnp.float32)
    zeros_c = jnp.zeros((N_PAD, H), jnp.float32)
    ones_r = jnp.ones((CHUNK, 16), jnp.float32)

    degp = _deg_kernel(eidx3, zeros_d, ones_r).reshape(2, HALF, 32)
    hs1 = _tc1(x, W1, degp)
    p1 = _conv_kernel(hs1.reshape(N_PAD, H), eidx3, zeros_c)
    hs2 = _tc2(p1.reshape(2, HALF, 128), hs1, degp, W2, b1, g1, bt1)
    p2 = _conv_kernel(hs2.reshape(N_PAD, H), eidx3, zeros_c)
    out = _tc3(p2.reshape(2, HALF, 128), hs2, degp, b2, g2, bt2,
               mW1, mb1, mW2, mb2, mW3, mb3)
    return out.reshape(N)


# speedup vs baseline: 55.5558x; 1.1409x over previous
import functools

import jax
import jax.numpy as jnp
from jax import lax
from jax.experimental import pallas as pl
from jax.experimental.pallas import tpu as pltpu
from jax.experimental.pallas import tpu_sc as plsc

N = 10000
D_IN = 128
H = 64

NC = 2
NS = 16
NW = NC * NS

N_PAD = 10240
ROWS_PER_SUB = N_PAD // NS
CHUNK = 128
CHUNKS_PER_W = 80
EW = CHUNK * CHUNKS_PER_W
E_PAD = EW * NW

_mesh = plsc.VectorSubcoreMesh(core_axis_name="c", subcore_axis_name="s")
_sc_params = pltpu.CompilerParams(use_tc_tiling_on_sc=False)


@functools.partial(
    pl.kernel,
    out_type=jax.ShapeDtypeStruct((NC, N_PAD, 16), jnp.float32),
    mesh=_mesh,
    scratch_types=[
        pltpu.VMEM((CHUNKS_PER_W, CHUNK), jnp.int32),
        pltpu.VMEM((CHUNK, 16), jnp.float32),
        pltpu.VMEM_SHARED((N_PAD, 16), jnp.float32),
        [pltpu.SemaphoreType.DMA] * 8,
    ],
    compiler_params=_sc_params,
)
def _deg_kernel(eidx_hbm, zeros_hbm, ones_hbm, out_hbm, didx, ones_v, acc, sems):
    c = lax.axis_index("c")
    s = lax.axis_index("s")
    wid = s * NC + c
    rbase = s * ROWS_PER_SUB
    pltpu.sync_copy(ones_hbm, ones_v)
    pltpu.sync_copy(eidx_hbm.at[1, pl.ds(wid * CHUNKS_PER_W, CHUNKS_PER_W)],
                    didx)
    pltpu.sync_copy(zeros_hbm.at[pl.ds(rbase, ROWS_PER_SUB)],
                    acc.at[pl.ds(rbase, ROWS_PER_SUB)])
    plsc.subcore_barrier()

    def body(k, carry):
        base = k * 8
        cps = [pltpu.async_copy(ones_v, acc.at[didx.at[base + b]], sems[b],
                                add=True) for b in range(8)]
        for cp in cps:
            cp.wait()
        return carry

    lax.fori_loop(0, CHUNKS_PER_W // 8, body, 0)
    plsc.subcore_barrier()
    pltpu.sync_copy(acc.at[pl.ds(rbase, ROWS_PER_SUB)],
                    out_hbm.at[c, pl.ds(rbase, ROWS_PER_SUB)])


@functools.partial(
    pl.kernel,
    out_type=jax.ShapeDtypeStruct((NC, N_PAD, H), jnp.bfloat16),
    mesh=_mesh,
    scratch_types=[
        pltpu.VMEM((CHUNKS_PER_W, CHUNK), jnp.int32),
        pltpu.VMEM((CHUNKS_PER_W, CHUNK), jnp.int32),
        pltpu.VMEM((8, CHUNK, H), jnp.bfloat16),
        pltpu.VMEM_SHARED((N_PAD, H), jnp.bfloat16),
        [pltpu.SemaphoreType.DMA] * 8,
        [pltpu.SemaphoreType.DMA] * 8,
    ],
    compiler_params=_sc_params,
)
def _conv_kernel(hs_hbm, eidx_hbm, zeros_hbm, out_hbm,
                 sidx, didx, rows, acc, gsems, ssems):
    c = lax.axis_index("c")
    s = lax.axis_index("s")
    wid = s * NC + c
    rbase = s * ROWS_PER_SUB
    pltpu.sync_copy(eidx_hbm.at[0, pl.ds(wid * CHUNKS_PER_W, CHUNKS_PER_W)],
                    sidx)
    pltpu.sync_copy(eidx_hbm.at[1, pl.ds(wid * CHUNKS_PER_W, CHUNKS_PER_W)],
                    didx)
    pltpu.sync_copy(zeros_hbm.at[pl.ds(rbase, ROWS_PER_SUB)],
                    acc.at[pl.ds(rbase, ROWS_PER_SUB)])
    plsc.subcore_barrier()

    def gather(j, b):
        return pltpu.async_copy(hs_hbm.at[sidx.at[j]], rows.at[b], gsems[b])

    def scatter(j, b):
        return pltpu.async_copy(rows.at[b], acc.at[didx.at[j]], ssems[b],
                                add=True)

    cg_pro = [gather(b, b) for b in range(4)]

    def body(k, carry):
        base = k * 8
        cs = []
        for b in range(4):
            pltpu.make_async_copy(hs_hbm.at[sidx.at[base + b]], rows.at[b],
                                  gsems[b]).wait()
            cs.append(scatter(base + b, b))
        cg2 = [gather(base + 4 + b, 4 + b) for b in range(4)]
        for cp in cs:
            cp.wait()
        cs2 = []
        for b in range(4):
            cg2[b].wait()
            cs2.append(scatter(base + 4 + b, 4 + b))
        for b in range(4):
            jn = jnp.minimum(base + 8 + b, CHUNKS_PER_W - 1)
            gather(jn, b)
        for cp in cs2:
            cp.wait()
        return carry

    lax.fori_loop(0, CHUNKS_PER_W // 8, body, 0)
    for b in range(4):
        pltpu.make_async_copy(hs_hbm.at[sidx.at[CHUNKS_PER_W - 1]],
                              rows.at[b], gsems[b]).wait()
    plsc.subcore_barrier()
    pltpu.sync_copy(acc.at[pl.ds(rbase, ROWS_PER_SUB)],
                    out_hbm.at[c, pl.ds(rbase, ROWS_PER_SUB)])


_BN_C = 1.0 / (1.0 + 1e-5) ** 0.5
HALF = N_PAD // 2
HI_N = N - HALF


def _dinv_from(degp_ref):
    deg_lo = degp_ref[0, :, 0:1] + degp_ref[1, :, 0:1] + 1.0
    deg_hi = degp_ref[0, :HI_N, 16:17] + degp_ref[1, :HI_N, 16:17] + 1.0
    return lax.rsqrt(deg_lo), lax.rsqrt(deg_hi)


def _tc1_body(x_ref, w1_ref, degp_ref, hs_ref):
    dlo, dhi = _dinv_from(degp_ref)
    y_lo = jnp.dot(x_ref[:HALF], w1_ref[...],
                   preferred_element_type=jnp.float32)
    y_hi = jnp.dot(x_ref[HALF:], w1_ref[...],
                   preferred_element_type=jnp.float32)
    hs_ref[:, :H] = (y_lo * dlo).astype(jnp.bfloat16)
    hs_ref[:HI_N, H:] = (y_hi * dhi).astype(jnp.bfloat16)
    hs_ref[HI_N:, H:] = jnp.zeros((HALF - HI_N, H), jnp.bfloat16)


def _sum_pk(p_ref, hs_ref):
    return (p_ref[0].astype(jnp.float32) + p_ref[1].astype(jnp.float32)
            + hs_ref[...].astype(jnp.float32))


def _tc2_body(p_ref, hs_ref, degp_ref, w2_ref, b1_ref, g1_ref, bt1_ref,
              hs2_ref):
    dlo, dhi = _dinv_from(degp_ref)
    spk = _sum_pk(p_ref, hs_ref)
    c1 = g1_ref[...] * _BN_C
    h1_lo = jax.nn.relu((dlo * spk[:, :H] + b1_ref[...]) * c1 + bt1_ref[...])
    h1_hi = jax.nn.relu((dhi * spk[:HI_N, H:] + b1_ref[...]) * c1
                        + bt1_ref[...])
    y2_lo = jnp.dot(h1_lo, w2_ref[...], preferred_element_type=jnp.float32)
    y2_hi = jnp.dot(h1_hi, w2_ref[...], preferred_element_type=jnp.float32)
    hs2_ref[:, :H] = (y2_lo * dlo).astype(jnp.bfloat16)
    hs2_ref[:HI_N, H:] = (y2_hi * dhi).astype(jnp.bfloat16)
    hs2_ref[HI_N:, H:] = jnp.zeros((HALF - HI_N, H), jnp.bfloat16)


def _tc3_body(p_ref, hs_ref, degp_ref, b2_ref, g2_ref, bt2_ref,
              mw1_ref, mb1_ref, mw2_ref, mb2_ref, mw3_ref, mb3_ref, out_ref):
    dlo, dhi = _dinv_from(degp_ref)
    spk = _sum_pk(p_ref, hs_ref)
    c2 = g2_ref[...] * _BN_C

    def head(h2):
        m1 = jax.nn.relu(
            jnp.dot(h2, mw1_ref[...], preferred_element_type=jnp.float32)
            + mb1_ref[...])
        m2 = jax.nn.relu(
            jnp.dot(m1, mw2_ref[...], preferred_element_type=jnp.float32)
            + mb2_ref[...])
        z = (jnp.dot(m2, mw3_ref[...], preferred_element_type=jnp.float32)
             + mb3_ref[...])
        return jax.nn.sigmoid(z)

    h2_lo = jax.nn.relu((dlo * spk[:, :H] + b2_ref[...]) * c2 + bt2_ref[...])
    h2_hi = jax.nn.relu((dhi * spk[:HI_N, H:] + b2_ref[...]) * c2
                        + bt2_ref[...])
    out_ref[:HALF] = head(h2_lo)[:, 0]
    out_ref[HALF:] = head(h2_hi)[:, 0]


_tc1 = pl.pallas_call(
    _tc1_body, out_shape=jax.ShapeDtypeStruct((HALF, 128), jnp.bfloat16))
_tc2 = pl.pallas_call(
    _tc2_body, out_shape=jax.ShapeDtypeStruct((HALF, 128), jnp.bfloat16))
_tc3 = pl.pallas_call(
    _tc3_body, out_shape=jax.ShapeDtypeStruct((N,), jnp.float32))


def kernel(x, edge_index, W1, b1, W2, b2, g1, bt1, g2, bt2,
           mW1, mb1, mW2, mb2, mW3, mb3):
    E = edge_index.shape[1]
    pad = E_PAD - E
    eperm = jnp.where(edge_index < HALF, 2 * edge_index,
                      2 * (edge_index - HALF) + 1)
    pad_ar = jnp.arange(pad, dtype=jnp.int32)
    pads = jnp.stack([pad_ar % 1024,
                      2 * (N - HALF) + 1 + 2 * (pad_ar % (N_PAD - N))])
    eidx3 = jnp.concatenate([eperm, pads], axis=1).reshape(
        2, E_PAD // CHUNK, CHUNK)

    zeros_d = jnp.zeros((N_PAD, 16), jnp.float32)
    zeros_c = jnp.zeros((N_PAD, H), jnp.bfloat16)
    ones_r = jnp.ones((CHUNK, 16), jnp.float32)

    degp = _deg_kernel(eidx3, zeros_d, ones_r).reshape(2, HALF, 32)
    hs1 = _tc1(x, W1, degp)
    p1 = _conv_kernel(hs1.reshape(N_PAD, H), eidx3, zeros_c)
    hs2 = _tc2(p1.reshape(2, HALF, 128), hs1, degp, W2, b1, g1, bt1)
    p2 = _conv_kernel(hs2.reshape(N_PAD, H), eidx3, zeros_c)
    return _tc3(p2.reshape(2, HALF, 128), hs2, degp, b2, g2, bt2,
                mW1, mb1, mW2, mb2, mW3, mb3)


# speedup vs baseline: 57.3811x; 1.0329x over previous
import functools

import jax
import jax.numpy as jnp
from jax import lax
from jax.experimental import pallas as pl
from jax.experimental.pallas import tpu as pltpu
from jax.experimental.pallas import tpu_sc as plsc

N = 10000
D_IN = 128
H = 64

NC = 2
NS = 16
NW = NC * NS

N_PAD = 10240
ROWS_PER_SUB = N_PAD // NS
CHUNK = 128
CHUNKS_PER_W = 80
EW = CHUNK * CHUNKS_PER_W
E_PAD = EW * NW

_mesh = plsc.VectorSubcoreMesh(core_axis_name="c", subcore_axis_name="s")
_sc_params = pltpu.CompilerParams(use_tc_tiling_on_sc=False)


NCH_RAW = 320000 // CHUNK
DEG_CPW = NCH_RAW // NW


@functools.partial(
    pl.kernel,
    out_type=jax.ShapeDtypeStruct((NC, N_PAD, 16), jnp.float32),
    mesh=_mesh,
    scratch_types=[
        pltpu.VMEM((DEG_CPW + 1, CHUNK), jnp.int32),
        pltpu.VMEM((CHUNK, 16), jnp.float32),
        pltpu.VMEM_SHARED((N_PAD, 16), jnp.float32),
        [pltpu.SemaphoreType.DMA] * 8,
    ],
    compiler_params=_sc_params,
)
def _deg_kernel(eidx_hbm, zeros_hbm, ones_hbm, out_hbm, didx, ones_v, acc, sems):
    c = lax.axis_index("c")
    s = lax.axis_index("s")
    wid = s * NC + c
    rbase = s * ROWS_PER_SUB
    extra = wid < (NCH_RAW - DEG_CPW * NW)
    pltpu.sync_copy(ones_hbm, ones_v)
    pltpu.sync_copy(eidx_hbm.at[1, pl.ds(wid * DEG_CPW, DEG_CPW)],
                    didx.at[pl.ds(0, DEG_CPW)])

    @pl.when(extra)
    def _():
        pltpu.sync_copy(eidx_hbm.at[1, pl.ds(DEG_CPW * NW + wid, 1)],
                        didx.at[pl.ds(DEG_CPW, 1)])

    pltpu.sync_copy(zeros_hbm.at[pl.ds(rbase, ROWS_PER_SUB)],
                    acc.at[pl.ds(rbase, ROWS_PER_SUB)])
    nch = DEG_CPW + extra.astype(jnp.int32)

    def permute(j, carry):
        for g in range(CHUNK // 16):
            v = didx[j, pl.ds(g * 16, 16)]
            didx[j, pl.ds(g * 16, 16)] = jnp.where(
                v < HALF, v + v, v + v - (2 * HALF - 1))
        return carry

    lax.fori_loop(0, nch, permute, 0)
    plsc.subcore_barrier()

    def body(k, carry):
        base = k * 8
        cps = [pltpu.async_copy(ones_v, acc.at[didx.at[base + b]], sems[b],
                                add=True) for b in range(8)]
        for cp in cps:
            cp.wait()
        return carry

    lax.fori_loop(0, DEG_CPW // 8, body, 0)
    tail = [pltpu.async_copy(ones_v, acc.at[didx.at[(DEG_CPW // 8) * 8 + b]],
                             sems[b], add=True)
            for b in range(DEG_CPW % 8)]
    for cp in tail:
        cp.wait()

    @pl.when(extra)
    def _():
        pltpu.sync_copy(ones_v, acc.at[didx.at[DEG_CPW]], add=True)

    plsc.subcore_barrier()
    pltpu.sync_copy(acc.at[pl.ds(rbase, ROWS_PER_SUB)],
                    out_hbm.at[c, pl.ds(rbase, ROWS_PER_SUB)])


@functools.partial(
    pl.kernel,
    out_type=jax.ShapeDtypeStruct((NC, N_PAD, H), jnp.bfloat16),
    mesh=_mesh,
    scratch_types=[
        pltpu.VMEM((CHUNKS_PER_W, CHUNK), jnp.int32),
        pltpu.VMEM((CHUNKS_PER_W, CHUNK), jnp.int32),
        pltpu.VMEM((8, CHUNK, H), jnp.bfloat16),
        pltpu.VMEM_SHARED((N_PAD, H), jnp.bfloat16),
        [pltpu.SemaphoreType.DMA] * 8,
        [pltpu.SemaphoreType.DMA] * 8,
    ],
    compiler_params=_sc_params,
)
def _conv_kernel(hs_hbm, eidx_hbm, zeros_hbm, out_hbm,
                 sidx, didx, rows, acc, gsems, ssems):
    c = lax.axis_index("c")
    s = lax.axis_index("s")
    wid = s * NC + c
    rbase = s * ROWS_PER_SUB
    pltpu.sync_copy(eidx_hbm.at[0, pl.ds(wid * CHUNKS_PER_W, CHUNKS_PER_W)],
                    sidx)
    pltpu.sync_copy(eidx_hbm.at[1, pl.ds(wid * CHUNKS_PER_W, CHUNKS_PER_W)],
                    didx)
    pltpu.sync_copy(zeros_hbm.at[pl.ds(rbase, ROWS_PER_SUB)],
                    acc.at[pl.ds(rbase, ROWS_PER_SUB)])
    plsc.subcore_barrier()

    def gather(j, b):
        return pltpu.async_copy(hs_hbm.at[sidx.at[j]], rows.at[b], gsems[b])

    def scatter(j, b):
        return pltpu.async_copy(rows.at[b], acc.at[didx.at[j]], ssems[b],
                                add=True)

    cg_pro = [gather(b, b) for b in range(4)]

    def body(k, carry):
        base = k * 8
        cs = []
        for b in range(4):
            pltpu.make_async_copy(hs_hbm.at[sidx.at[base + b]], rows.at[b],
                                  gsems[b]).wait()
            cs.append(scatter(base + b, b))
        cg2 = [gather(base + 4 + b, 4 + b) for b in range(4)]
        for cp in cs:
            cp.wait()
        cs2 = []
        for b in range(4):
            cg2[b].wait()
            cs2.append(scatter(base + 4 + b, 4 + b))
        for b in range(4):
            jn = jnp.minimum(base + 8 + b, CHUNKS_PER_W - 1)
            gather(jn, b)
        for cp in cs2:
            cp.wait()
        return carry

    lax.fori_loop(0, CHUNKS_PER_W // 8, body, 0)
    for b in range(4):
        pltpu.make_async_copy(hs_hbm.at[sidx.at[CHUNKS_PER_W - 1]],
                              rows.at[b], gsems[b]).wait()
    plsc.subcore_barrier()
    pltpu.sync_copy(acc.at[pl.ds(rbase, ROWS_PER_SUB)],
                    out_hbm.at[c, pl.ds(rbase, ROWS_PER_SUB)])


_BN_C = 1.0 / (1.0 + 1e-5) ** 0.5
HALF = N_PAD // 2
HI_N = N - HALF


def _dinv_from(degp_ref):
    deg_lo = degp_ref[0, :, 0:1] + degp_ref[1, :, 0:1] + 1.0
    deg_hi = degp_ref[0, :HI_N, 16:17] + degp_ref[1, :HI_N, 16:17] + 1.0
    return lax.rsqrt(deg_lo), lax.rsqrt(deg_hi)


def _tc1_body(x_ref, w1_ref, degp_ref, hs_ref):
    dlo, dhi = _dinv_from(degp_ref)
    y_lo = jnp.dot(x_ref[:HALF], w1_ref[...],
                   preferred_element_type=jnp.float32)
    y_hi = jnp.dot(x_ref[HALF:], w1_ref[...],
                   preferred_element_type=jnp.float32)
    hs_ref[:, :H] = (y_lo * dlo).astype(jnp.bfloat16)
    hs_ref[:HI_N, H:] = (y_hi * dhi).astype(jnp.bfloat16)
    hs_ref[HI_N:, H:] = jnp.zeros((HALF - HI_N, H), jnp.bfloat16)


def _sum_pk(p_ref, hs_ref):
    return (p_ref[0].astype(jnp.float32) + p_ref[1].astype(jnp.float32)
            + hs_ref[...].astype(jnp.float32))


def _tc2_body(p_ref, hs_ref, degp_ref, w2_ref, b1_ref, g1_ref, bt1_ref,
              hs2_ref):
    dlo, dhi = _dinv_from(degp_ref)
    spk = _sum_pk(p_ref, hs_ref)
    c1 = g1_ref[...] * _BN_C
    h1_lo = jax.nn.relu((dlo * spk[:, :H] + b1_ref[...]) * c1 + bt1_ref[...])
    h1_hi = jax.nn.relu((dhi * spk[:HI_N, H:] + b1_ref[...]) * c1
                        + bt1_ref[...])
    y2_lo = jnp.dot(h1_lo, w2_ref[...], preferred_element_type=jnp.float32)
    y2_hi = jnp.dot(h1_hi, w2_ref[...], preferred_element_type=jnp.float32)
    hs2_ref[:, :H] = (y2_lo * dlo).astype(jnp.bfloat16)
    hs2_ref[:HI_N, H:] = (y2_hi * dhi).astype(jnp.bfloat16)
    hs2_ref[HI_N:, H:] = jnp.zeros((HALF - HI_N, H), jnp.bfloat16)


def _tc3_body(p_ref, hs_ref, degp_ref, b2_ref, g2_ref, bt2_ref,
              mw1_ref, mb1_ref, mw2_ref, mb2_ref, mw3_ref, mb3_ref, out_ref):
    dlo, dhi = _dinv_from(degp_ref)
    spk = _sum_pk(p_ref, hs_ref)
    c2 = g2_ref[...] * _BN_C

    def head(h2):
        m1 = jax.nn.relu(
            jnp.dot(h2, mw1_ref[...], preferred_element_type=jnp.float32)
            + mb1_ref[...])
        m2 = jax.nn.relu(
            jnp.dot(m1, mw2_ref[...], preferred_element_type=jnp.float32)
            + mb2_ref[...])
        z = (jnp.dot(m2, mw3_ref[...], preferred_element_type=jnp.float32)
             + mb3_ref[...])
        return jax.nn.sigmoid(z)

    h2_lo = jax.nn.relu((dlo * spk[:, :H] + b2_ref[...]) * c2 + bt2_ref[...])
    h2_hi = jax.nn.relu((dhi * spk[:HI_N, H:] + b2_ref[...]) * c2
                        + bt2_ref[...])
    out_ref[:HALF] = head(h2_lo)[:, 0]
    out_ref[HALF:] = head(h2_hi)[:, 0]


_tc1 = pl.pallas_call(
    _tc1_body, out_shape=jax.ShapeDtypeStruct((HALF, 128), jnp.bfloat16))
_tc2 = pl.pallas_call(
    _tc2_body, out_shape=jax.ShapeDtypeStruct((HALF, 128), jnp.bfloat16))
_tc3 = pl.pallas_call(
    _tc3_body, out_shape=jax.ShapeDtypeStruct((N,), jnp.float32))


def kernel(x, edge_index, W1, b1, W2, b2, g1, bt1, g2, bt2,
           mW1, mb1, mW2, mb2, mW3, mb3):
    E = edge_index.shape[1]
    pad = E_PAD - E
    eperm = jnp.where(edge_index < HALF, 2 * edge_index,
                      2 * (edge_index - HALF) + 1)
    pad_ar = jnp.arange(pad, dtype=jnp.int32)
    pads = jnp.stack([pad_ar % 1024,
                      2 * (N - HALF) + 1 + 2 * (pad_ar % (N_PAD - N))])
    eidx3 = jnp.concatenate([eperm, pads], axis=1).reshape(
        2, E_PAD // CHUNK, CHUNK)

    zeros_d = jnp.zeros((N_PAD, 16), jnp.float32)
    zeros_c = jnp.zeros((N_PAD, H), jnp.bfloat16)
    ones_r = jnp.ones((CHUNK, 16), jnp.float32)

    eidx_raw = edge_index.reshape(2, NCH_RAW, CHUNK)
    degp = _deg_kernel(eidx_raw, zeros_d, ones_r).reshape(2, HALF, 32)
    hs1 = _tc1(x, W1, degp)
    p1 = _conv_kernel(hs1.reshape(N_PAD, H), eidx3, zeros_c)
    hs2 = _tc2(p1.reshape(2, HALF, 128), hs1, degp, W2, b1, g1, bt1)
    p2 = _conv_kernel(hs2.reshape(N_PAD, H), eidx3, zeros_c)
    return _tc3(p2.reshape(2, HALF, 128), hs2, degp, b2, g2, bt2,
                mW1, mb1, mW2, mb2, mW3, mb3)
